# baseline (speedup 1.0000x reference)
import functools

import jax
import jax.numpy as jnp
from jax import lax
from jax.experimental import pallas as pl
from jax.experimental.pallas import tpu as pltpu
from jax.experimental.pallas import tpu_sc as plsc

VOCAB = 1000000
EMBED_DIM = 32
BATCH = 4096
MAX_LEN = 200

_INFO = plsc.get_sparse_core_info()
_NC = _INFO.num_cores
_NS = _INFO.num_subcores
_NW = _NC * _NS

_BB = BATCH // _NW
_DT = EMBED_DIM // 8
_CH = 4
_NCH = MAX_LEN // _CH


def _make_kernel():
    mesh = plsc.VectorSubcoreMesh(core_axis_name="c", subcore_axis_name="s")

    @functools.partial(
        pl.kernel,
        mesh=mesh,
        compiler_params=pltpu.CompilerParams(
            use_tc_tiling_on_sc=False, needs_layout_passes=False
        ),
        out_type=jax.ShapeDtypeStruct((MAX_LEN, _DT, _NW, 8 * _BB), jnp.float32),
        scratch_types=[
            pltpu.VMEM((MAX_LEN, _BB), jnp.int32),
            pltpu.VMEM((_CH * _BB, EMBED_DIM), jnp.float32),
            pltpu.VMEM((_CH * _BB, EMBED_DIM), jnp.float32),
            pltpu.VMEM((_DT, 8 * _BB), jnp.float32),
            pltpu.VMEM((_DT, 8 * _BB), jnp.float32),
            pltpu.VMEM((_DT, 8 * _BB), jnp.float32),
            pltpu.VMEM((_DT, 8 * _BB), jnp.float32),
            pltpu.SemaphoreType.DMA,
            pltpu.SemaphoreType.DMA,
            pltpu.SemaphoreType.DMA,
            pltpu.SemaphoreType.DMA,
            pltpu.SemaphoreType.DMA,
            pltpu.SemaphoreType.DMA,
        ],
    )
    def emb_kernel(idx_hbm, table_hbm, out_hbm, idx_v, buf0, buf1,
                   bta, btb, btc, btd, sg0, sg1, ssa, ssb, ssc, ssd):
        wid = lax.axis_index("s") * _NC + lax.axis_index("c")
        pltpu.sync_copy(idx_hbm.at[wid], idx_v)
        lanes = lax.iota(jnp.int32, 16)
        dts = [(d0 + lanes) >> 3 for d0 in (0, 16)]
        ibase = [((d0 + lanes) & 7) * _BB for d0 in (0, 16)]
        bts = (bta, btb, btc, btd)
        sss = (ssa, ssb, ssc, ssd)

        def fire(c, buf, sem):
            for k in range(_CH):
                pltpu.async_copy(
                    table_hbm.at[idx_v.at[c * _CH + k]],
                    buf.at[pl.ds(k * _BB, _BB)],
                    sem,
                )

        def drain_gather(buf, sem):
            pltpu.make_async_copy(
                table_hbm.at[pl.ds(0, _CH * _BB)], buf, sem
            ).wait()

        def transpose(buf, k, bt):
            def tbody(g, _):
                for kk in range(8):
                    bb = g * 8 + kk
                    row = k * _BB + bb
                    for h in range(2):
                        vals = buf[row, pl.ds(h * 16, 16)]
                        plsc.store_scatter(bt, [dts[h], ibase[h] + bb], vals)
                return 0

            lax.fori_loop(0, _BB // 8, tbody, 0)

        def store_start(bt, l, sem):
            pltpu.async_copy(bt, out_hbm.at[l, :, wid, :], sem)

        def store_wait(bt, sem):
            pltpu.make_async_copy(bt, out_hbm.at[0, :, 0, :], sem).wait()

        def proc(c, buf, sg, first=False):
            drain_gather(buf, sg)
            for k in range(_CH):
                if not first:
                    store_wait(bts[k], sss[k])
                transpose(buf, k, bts[k])
                store_start(bts[k], c * _CH + k, sss[k])

        fire(0, buf0, sg0)
        fire(1, buf1, sg1)
        proc(0, buf0, sg0, first=True)
        fire(2, buf0, sg0)
        proc(1, buf1, sg1)
        fire(3, buf1, sg1)

        def pair(p, _):
            c0 = 2 * p
            proc(c0, buf0, sg0)
            fire(c0 + 2, buf0, sg0)
            proc(c0 + 1, buf1, sg1)
            fire(c0 + 3, buf1, sg1)
            return 0

        lax.fori_loop(1, _NCH // 2 - 1, pair, 0)

        proc(_NCH - 2, buf0, sg0)
        proc(_NCH - 1, buf1, sg1)
        for k in range(_CH):
            store_wait(bts[k], sss[k])

    return emb_kernel


_EMB_KERNEL = _make_kernel()


@jax.jit
def kernel(inputs, table):
    idx = inputs.astype(jnp.int32).T.reshape(MAX_LEN, _NW, _BB).transpose(1, 0, 2)
    out5 = _EMB_KERNEL(idx, table).reshape(MAX_LEN, _DT, _NW, 8, _BB)
    return out5.transpose(2, 4, 0, 1, 3).reshape(BATCH, MAX_LEN, EMBED_DIM)


# speedup vs baseline: 1.1612x; 1.1612x over previous
import functools

import jax
import jax.numpy as jnp
from jax import lax
from jax.experimental import pallas as pl
from jax.experimental.pallas import tpu as pltpu
from jax.experimental.pallas import tpu_sc as plsc

VOCAB = 1000000
EMBED_DIM = 32
BATCH = 4096
MAX_LEN = 200

_INFO = plsc.get_sparse_core_info()
_NC = _INFO.num_cores
_NS = _INFO.num_subcores
_NW = _NC * _NS

_BB = BATCH // _NW
_DT = EMBED_DIM // 8
_CH = 4
_NCH = MAX_LEN // _CH


def _make_kernel():
    mesh = plsc.VectorSubcoreMesh(core_axis_name="c", subcore_axis_name="s")

    @functools.partial(
        pl.kernel,
        mesh=mesh,
        compiler_params=pltpu.CompilerParams(
            use_tc_tiling_on_sc=False, needs_layout_passes=False
        ),
        out_type=jax.ShapeDtypeStruct((MAX_LEN, _DT, _NW, 8 * _BB), jnp.float32),
        scratch_types=[
            pltpu.VMEM((MAX_LEN, _BB), jnp.int32),
            pltpu.VMEM((_CH * _BB, EMBED_DIM), jnp.float32),
            pltpu.VMEM((_CH * _BB, EMBED_DIM), jnp.float32),
            pltpu.VMEM((_DT, 8 * _BB), jnp.float32),
            pltpu.VMEM((_DT, 8 * _BB), jnp.float32),
            pltpu.VMEM((_DT, 8 * _BB), jnp.float32),
            pltpu.VMEM((_DT, 8 * _BB), jnp.float32),
            pltpu.SemaphoreType.DMA,
            pltpu.SemaphoreType.DMA,
            pltpu.SemaphoreType.DMA,
            pltpu.SemaphoreType.DMA,
            pltpu.SemaphoreType.DMA,
            pltpu.SemaphoreType.DMA,
        ],
    )
    def emb_kernel(idx_hbm, table_hbm, out_hbm, idx_v, buf0, buf1,
                   bta, btb, btc, btd, sg0, sg1, ssa, ssb, ssc, ssd):
        wid = lax.axis_index("s") * _NC + lax.axis_index("c")
        pltpu.sync_copy(idx_hbm.at[wid], idx_v)
        lanes = lax.iota(jnp.int32, 16)
        dts = [(d0 + lanes) >> 3 for d0 in (0, 16)]
        ibase = [((d0 + lanes) & 7) * _BB for d0 in (0, 16)]
        bts = (bta, btb, btc, btd)
        sss = (ssa, ssb, ssc, ssd)

        def fire(c, buf, sem):
            for k in range(_CH):
                pltpu.async_copy(
                    table_hbm.at[idx_v.at[c * _CH + k]],
                    buf.at[pl.ds(k * _BB, _BB)],
                    sem,
                )

        def drain_gather(buf, sem):
            pltpu.make_async_copy(
                table_hbm.at[pl.ds(0, _CH * _BB)], buf, sem
            ).wait()

        def transpose(buf, k, bt):
            @plsc.parallel_loop(0, _BB, step=8, unroll=2)
            def tbody(g):
                for kk in range(8):
                    bb = g + kk
                    row = k * _BB + bb
                    for h in range(2):
                        vals = buf[row, pl.ds(h * 16, 16)]
                        plsc.store_scatter(bt, [dts[h], ibase[h] + bb], vals)

        def store_start(bt, l, sem):
            pltpu.async_copy(bt, out_hbm.at[l, :, wid, :], sem)

        def store_wait(bt, sem):
            pltpu.make_async_copy(bt, out_hbm.at[0, :, 0, :], sem).wait()

        def proc(c, buf, sg, first=False):
            drain_gather(buf, sg)
            for k in range(_CH):
                if not first:
                    store_wait(bts[k], sss[k])
                transpose(buf, k, bts[k])
                store_start(bts[k], c * _CH + k, sss[k])

        fire(0, buf0, sg0)
        fire(1, buf1, sg1)
        proc(0, buf0, sg0, first=True)
        fire(2, buf0, sg0)
        proc(1, buf1, sg1)
        fire(3, buf1, sg1)

        def pair(p, _):
            c0 = 2 * p
            proc(c0, buf0, sg0)
            fire(c0 + 2, buf0, sg0)
            proc(c0 + 1, buf1, sg1)
            fire(c0 + 3, buf1, sg1)
            return 0

        lax.fori_loop(1, _NCH // 2 - 1, pair, 0)

        proc(_NCH - 2, buf0, sg0)
        proc(_NCH - 1, buf1, sg1)
        for k in range(_CH):
            store_wait(bts[k], sss[k])

    return emb_kernel


_EMB_KERNEL = _make_kernel()


@jax.jit
def kernel(inputs, table):
    idx = inputs.astype(jnp.int32).T.reshape(MAX_LEN, _NW, _BB).transpose(1, 0, 2)
    out5 = _EMB_KERNEL(idx, table).reshape(MAX_LEN, _DT, _NW, 8, _BB)
    return out5.transpose(2, 4, 0, 1, 3).reshape(BATCH, MAX_LEN, EMBED_DIM)
